# baseline (speedup 1.0000x reference)
import functools

import jax
import jax.numpy as jnp
from jax import lax
from jax.experimental import pallas as pl
from jax.experimental.pallas import tpu as pltpu
from jax.experimental.pallas import tpu_sc as plsc

D = 128
SCALE = float(128.0 ** 0.5)

NC = 2
NS = 16
NW = NC * NS

STEP = 256
NBUF = 2
W = 1


def _scale_body(t_ref, o_ref):
    o_ref[...] = (t_ref[...] * SCALE).astype(jnp.bfloat16)


def _scale_table(table):
    v, d = table.shape
    blk = 4000
    return pl.pallas_call(
        _scale_body,
        grid=(v // blk,),
        in_specs=[pl.BlockSpec((blk, d), lambda i: (i, 0))],
        out_specs=pl.BlockSpec((blk, d), lambda i: (i, 0)),
        out_shape=jax.ShapeDtypeStruct((v, d), jnp.bfloat16),
    )(table)


def _make_gather(B):
    assert B % (NW * STEP * NBUF) == 0
    steps = B // (NW * STEP)
    idx_per_w = steps * STEP
    outer = steps // NBUF

    mesh = plsc.VectorSubcoreMesh(core_axis_name="c", subcore_axis_name="s")

    @functools.partial(
        pl.kernel,
        mesh=mesh,
        compiler_params=pltpu.CompilerParams(use_tc_tiling_on_sc=False),
        out_type=jax.ShapeDtypeStruct((B, D), jnp.float32),
        scratch_types=[
            pltpu.VMEM((idx_per_w,), jnp.int32),
            pltpu.VMEM((NBUF, STEP, D // 2), jnp.int32),
            pltpu.VMEM((NBUF, STEP, D), jnp.float32),
        ] + [pltpu.SemaphoreType.DMA] * (2 * NBUF),
    )
    def gather(tab_hbm, idx_hbm, out_hbm, idx_v, raw_v, rows_v, *sems):
        sem_in = sems[:NBUF]
        sem_out = sems[NBUF:]
        wid = lax.axis_index("s") * NC + lax.axis_index("c")
        obase = wid * idx_per_w

        pltpu.sync_copy(idx_hbm.at[pl.ds(wid * idx_per_w, idx_per_w)], idx_v)

        def drain_out(q):
            pltpu.make_async_copy(
                out_hbm.at[pl.ds(0, STEP)], rows_v.at[q], sem_out[q]
            ).wait()

        def fire(s, q):
            pltpu.async_copy(
                tab_hbm.at[idx_v.at[pl.ds(s * STEP, STEP)]], raw_v.at[q], sem_in[q]
            )

        def expand(q):
            src = raw_v.at[q]
            dst = rows_v.at[q]
            hi = jnp.int32(-65536)

            def row2(r2, carry):
                r = r2 * 2
                for rr in range(2):
                    for c in range(D // 32):
                        u = src[r + rr, pl.ds(c * 16, 16)]
                        a = lax.bitcast_convert_type(u << 16, jnp.float32)
                        b = lax.bitcast_convert_type(u & hi, jnp.float32)
                        dst[r + rr, pl.ds(c * 32, 16)] = a
                        dst[r + rr, pl.ds(c * 32 + 16, 16)] = b
                return carry

            lax.fori_loop(0, STEP // 2, row2, 0)

        def retire(s, q):
            pltpu.make_async_copy(
                tab_hbm.at[pl.ds(0, STEP)], raw_v.at[q], sem_in[q]
            ).wait()
            expand(q)
            pltpu.async_copy(
                rows_v.at[q], out_hbm.at[pl.ds(obase + s * STEP, STEP)], sem_out[q]
            )

        def body(it, carry):
            for h in range(NBUF):
                s = it * NBUF + h
                @pl.when(it > 0)
                def _(h=h):
                    drain_out(h)
                fire(s, h)
                if h >= W:
                    retire(s - W, (h - W) % NBUF)
                else:
                    @pl.when(it > 0)
                    def _(s=s, h=h):
                        retire(s - W, (h - W) % NBUF)
            return carry

        lax.fori_loop(0, outer, body, 0)
        for w in range(W, 0, -1):
            retire(steps - w, (steps - w) % NBUF)
        for q in range(NBUF):
            drain_out(q)

    return gather


def kernel(x, table):
    bsz, seq = x.shape
    B = bsz * seq
    v, d = table.shape
    tperm = table.reshape(v, d // 32, 2, 16).transpose(0, 1, 3, 2).reshape(v, d)
    scaled = _scale_table(tperm)
    packed = jax.lax.bitcast_convert_type(
        scaled.reshape(v, d // 2, 2), jnp.int32
    )
    idx = x.reshape(B).astype(jnp.int32)
    out = _make_gather(B)(packed, idx)
    return out.reshape(bsz, seq, D)


# speedup vs baseline: 1.3691x; 1.3691x over previous
import functools

import jax
import jax.numpy as jnp
from jax import lax
from jax.experimental import pallas as pl
from jax.experimental.pallas import tpu as pltpu
from jax.experimental.pallas import tpu_sc as plsc

D = 128
SCALE = float(128.0 ** 0.5)

NC = 2
NS = 16
NW = NC * NS

STEP = 256
NBUF = 2
W = 1


def _scale_body(t_ref, o_ref):
    o_ref[...] = (t_ref[...] * SCALE).astype(jnp.bfloat16)


def _scale_table(table):
    v, d = table.shape
    blk = 4000
    return pl.pallas_call(
        _scale_body,
        grid=(v // blk,),
        in_specs=[pl.BlockSpec((blk, d), lambda i: (i, 0))],
        out_specs=pl.BlockSpec((blk, d), lambda i: (i, 0)),
        out_shape=jax.ShapeDtypeStruct((v, d), jnp.bfloat16),
    )(table)


def _make_gather(B):
    assert B % (NW * STEP * NBUF) == 0
    steps = B // (NW * STEP)
    idx_per_w = steps * STEP
    outer = steps // NBUF

    mesh = plsc.VectorSubcoreMesh(core_axis_name="c", subcore_axis_name="s")

    @functools.partial(
        pl.kernel,
        mesh=mesh,
        compiler_params=pltpu.CompilerParams(use_tc_tiling_on_sc=False),
        out_type=jax.ShapeDtypeStruct((B, D), jnp.float32),
        scratch_types=[
            pltpu.VMEM((idx_per_w,), jnp.int32),
            pltpu.VMEM((NBUF, STEP, D // 2), jnp.int32),
            pltpu.VMEM((NBUF, STEP, D), jnp.float32),
        ] + [pltpu.SemaphoreType.DMA] * (2 * NBUF),
    )
    def gather(tab_hbm, idx_hbm, out_hbm, idx_v, raw_v, rows_v, *sems):
        sem_in = sems[:NBUF]
        sem_out = sems[NBUF:]
        wid = lax.axis_index("s") * NC + lax.axis_index("c")
        obase = wid * idx_per_w

        pltpu.sync_copy(idx_hbm.at[pl.ds(wid * idx_per_w, idx_per_w)], idx_v)

        def drain_out(q):
            pltpu.make_async_copy(
                out_hbm.at[pl.ds(0, STEP)], rows_v.at[q], sem_out[q]
            ).wait()

        def fire(s, q):
            pltpu.async_copy(
                tab_hbm.at[idx_v.at[pl.ds(s * STEP, STEP)]], raw_v.at[q], sem_in[q]
            )

        def expand(q):
            src = raw_v.at[q]
            dst = rows_v.at[q]
            hi = jnp.int32(-65536)

            @plsc.parallel_loop(0, STEP, unroll=8)
            def _row(r):
                for c in range(D // 32):
                    u = src[r, pl.ds(c * 16, 16)]
                    a = lax.bitcast_convert_type(u << 16, jnp.float32)
                    b = lax.bitcast_convert_type(u & hi, jnp.float32)
                    dst[r, pl.ds(c * 32, 16)] = a
                    dst[r, pl.ds(c * 32 + 16, 16)] = b

        def retire(s, q):
            pltpu.make_async_copy(
                tab_hbm.at[pl.ds(0, STEP)], raw_v.at[q], sem_in[q]
            ).wait()
            expand(q)
            pltpu.async_copy(
                rows_v.at[q], out_hbm.at[pl.ds(obase + s * STEP, STEP)], sem_out[q]
            )

        def body(it, carry):
            for h in range(NBUF):
                s = it * NBUF + h
                @pl.when(it > 0)
                def _(h=h):
                    drain_out(h)
                fire(s, h)
                if h >= W:
                    retire(s - W, (h - W) % NBUF)
                else:
                    @pl.when(it > 0)
                    def _(s=s, h=h):
                        retire(s - W, (h - W) % NBUF)
            return carry

        lax.fori_loop(0, outer, body, 0)
        for w in range(W, 0, -1):
            retire(steps - w, (steps - w) % NBUF)
        for q in range(NBUF):
            drain_out(q)

    return gather


def kernel(x, table):
    bsz, seq = x.shape
    B = bsz * seq
    v, d = table.shape
    tperm = table.reshape(v, d // 32, 2, 16).transpose(0, 1, 3, 2).reshape(v, d)
    scaled = _scale_table(tperm)
    packed = jax.lax.bitcast_convert_type(
        scaled.reshape(v, d // 2, 2), jnp.int32
    )
    idx = x.reshape(B).astype(jnp.int32)
    out = _make_gather(B)(packed, idx)
    return out.reshape(bsz, seq, D)


# speedup vs baseline: 3.7639x; 2.7492x over previous
import functools

import jax
import jax.numpy as jnp
from jax import lax
from jax.experimental import pallas as pl
from jax.experimental.pallas import tpu as pltpu
from jax.experimental.pallas import tpu_sc as plsc

D = 128
SCALE = float(128.0 ** 0.5)

NC = 2
NS = 16
NW = NC * NS

STEP = 128
NBUF = 5
W = 3


def _scale_body(t_ref, o_ref):
    o_ref[...] = t_ref[...] * SCALE


def _scale_table(table):
    v, d = table.shape
    blk = 4000
    return pl.pallas_call(
        _scale_body,
        grid=(v // blk,),
        in_specs=[pl.BlockSpec((blk, d), lambda i: (i, 0))],
        out_specs=pl.BlockSpec((blk, d), lambda i: (i, 0)),
        out_shape=jax.ShapeDtypeStruct((v, d), jnp.float32),
    )(table)


def _make_gather(B):
    assert B % (NW * STEP * NBUF) == 0
    steps = B // (NW * STEP)
    idx_rows_per_w = steps
    rows_per_w = steps * STEP
    outer = steps // NBUF

    mesh = plsc.VectorSubcoreMesh(core_axis_name="c", subcore_axis_name="s")

    @functools.partial(
        pl.kernel,
        mesh=mesh,
        out_type=jax.ShapeDtypeStruct((B, D), jnp.float32),
        scratch_types=[
            pltpu.VMEM((idx_rows_per_w, 128), jnp.int32),
            pltpu.VMEM((NBUF, STEP, D), jnp.float32),
        ] + [pltpu.SemaphoreType.DMA] * (2 * NBUF),
    )
    def gather(tab_hbm, idx_hbm, out_hbm, idx_v, rows_v, *sems):
        sem_in = sems[:NBUF]
        sem_out = sems[NBUF:]
        wid = lax.axis_index("s") * NC + lax.axis_index("c")
        obase = wid * rows_per_w

        pltpu.sync_copy(idx_hbm.at[pl.ds(wid * idx_rows_per_w, idx_rows_per_w)], idx_v)

        def drain_out(q):
            pltpu.make_async_copy(
                out_hbm.at[pl.ds(0, STEP)], rows_v.at[q], sem_out[q]
            ).wait()

        def fire(s, q):
            pltpu.async_copy(tab_hbm.at[idx_v.at[s]], rows_v.at[q], sem_in[q])

        def retire(s, q):
            pltpu.make_async_copy(
                tab_hbm.at[pl.ds(0, STEP)], rows_v.at[q], sem_in[q]
            ).wait()
            pltpu.async_copy(
                rows_v.at[q], out_hbm.at[pl.ds(obase + s * STEP, STEP)], sem_out[q]
            )

        def body(it, carry):
            for h in range(NBUF):
                s = it * NBUF + h
                @pl.when(it > 0)
                def _(h=h):
                    drain_out(h)
                fire(s, h)
                if h >= W:
                    retire(s - W, (h - W) % NBUF)
                else:
                    @pl.when(it > 0)
                    def _(s=s, h=h):
                        retire(s - W, (h - W) % NBUF)
            return carry

        lax.fori_loop(0, outer, body, 0)
        for w in range(W, 0, -1):
            retire(steps - w, (steps - w) % NBUF)
        for q in range(NBUF):
            drain_out(q)

    return gather


def kernel(x, table):
    bsz, seq = x.shape
    B = bsz * seq
    scaled = _scale_table(table)
    idx = x.reshape(B // 128, 128).astype(jnp.int32)
    out = _make_gather(B)(scaled, idx)
    return out.reshape(bsz, seq, D)


# speedup vs baseline: 4.1733x; 1.1088x over previous
import functools

import jax
import jax.numpy as jnp
from jax import lax
from jax.experimental import pallas as pl
from jax.experimental.pallas import tpu as pltpu
from jax.experimental.pallas import tpu_sc as plsc

D = 128
SCALE = float(128.0 ** 0.5)

NC = 2
NS = 16
NW = NC * NS

STEP = 128
NBUF = 5
W = 3


def _scale_body(t_ref, o_ref):
    o_ref[...] = t_ref[...] * SCALE


def _scale_table(table):
    v, d = table.shape
    blk = 4000
    return pl.pallas_call(
        _scale_body,
        grid=(v // blk,),
        in_specs=[pl.BlockSpec((blk, d), lambda i: (i, 0))],
        out_specs=pl.BlockSpec((blk, d), lambda i: (i, 0)),
        out_shape=jax.ShapeDtypeStruct((v, d), jnp.float32),
    )(table)


def _make_gather(B):
    assert B % (NW * STEP * NBUF) == 0
    steps = B // (NW * STEP)
    idx_rows_per_w = steps
    rows_per_w = steps * STEP
    outer = steps // NBUF

    mesh = plsc.VectorSubcoreMesh(core_axis_name="c", subcore_axis_name="s")

    @functools.partial(
        pl.kernel,
        mesh=mesh,
        out_type=jax.ShapeDtypeStruct((B, D), jnp.float32),
        scratch_types=[
            pltpu.VMEM((idx_rows_per_w, 128), jnp.int32),
            pltpu.VMEM((NBUF, STEP, D), jnp.float32),
        ] + [pltpu.SemaphoreType.DMA] * (2 * NBUF),
    )
    def gather(tab_hbm, idx_hbm, out_hbm, idx_v, rows_v, *sems):
        sem_in = sems[:NBUF]
        sem_out = sems[NBUF:]
        wid = lax.axis_index("s") * NC + lax.axis_index("c")
        obase = wid * rows_per_w

        pltpu.sync_copy(idx_hbm.at[pl.ds(wid * idx_rows_per_w, idx_rows_per_w)], idx_v)

        def drain_out(q):
            pltpu.make_async_copy(
                out_hbm.at[pl.ds(0, STEP)], rows_v.at[q], sem_out[q]
            ).wait()

        def fire(s, q):
            pltpu.async_copy(tab_hbm.at[idx_v.at[s]], rows_v.at[q], sem_in[q])

        def retire(s, q):
            pltpu.make_async_copy(
                tab_hbm.at[pl.ds(0, STEP)], rows_v.at[q], sem_in[q]
            ).wait()
            buf = rows_v.at[q]

            @plsc.parallel_loop(0, STEP, unroll=8)
            def _scale_row(r):
                for c in range(D // 16):
                    buf[r, pl.ds(c * 16, 16)] = buf[r, pl.ds(c * 16, 16)] * SCALE

            pltpu.async_copy(
                rows_v.at[q], out_hbm.at[pl.ds(obase + s * STEP, STEP)], sem_out[q]
            )

        def body(it, carry):
            for h in range(NBUF):
                s = it * NBUF + h
                @pl.when(it > 0)
                def _(h=h):
                    drain_out(h)
                fire(s, h)
                if h >= W:
                    retire(s - W, (h - W) % NBUF)
                else:
                    @pl.when(it > 0)
                    def _(s=s, h=h):
                        retire(s - W, (h - W) % NBUF)
            return carry

        lax.fori_loop(0, outer, body, 0)
        for w in range(W, 0, -1):
            retire(steps - w, (steps - w) % NBUF)
        for q in range(NBUF):
            drain_out(q)

    return gather


def kernel(x, table):
    bsz, seq = x.shape
    B = bsz * seq
    idx = x.reshape(B // 128, 128).astype(jnp.int32)
    out = _make_gather(B)(table, idx)
    return out.reshape(bsz, seq, D)
